# baseline (speedup 1.0000x reference)
import functools

import jax
import jax.numpy as jnp
from jax import lax
from jax.experimental import pallas as pl
from jax.experimental.pallas import tpu as pltpu
from jax.experimental.pallas import tpu_sc as plsc

HDIM = 256
NUMRELS = 16
NHEADS = 4
DH = HDIM // NHEADS
N = 10000
NPAD = 10240
E = 160000
BLK = 512
EPS = 1e-5

NB = 160
ROUNDS = 2
EB = 1600
NGRP = EB // 16
NCHUNK = E // EB
CAP = 16
BUF = 32
NEG = -1e30


def _dense_body(h_ref, relv_ref, WA1T_ref, WA2T_ref, WBT_ref, Wk1T_ref,
                Wk2T_ref, WqT_ref, sWA_ref, bA_ref, bB_ref,
                K_out, V_out, q_out, Ah_scr, stats_scr):
    r = pl.program_id(1)
    hb = h_ref[...]

    @pl.when(r == 0)
    def _precompute():
        Ah_scr[...] = jnp.dot(hb, WA1T_ref[...],
                              preferred_element_type=jnp.float32)
        stats_scr[:, 0:1] = jnp.sum(hb, axis=1, keepdims=True)
        stats_scr[:, 1:2] = jnp.sum(hb * hb, axis=1, keepdims=True)
        q_out[...] = jnp.dot(hb, WqT_ref[...],
                             preferred_element_type=jnp.float32)

    rrow = relv_ref[0]
    Ar = jnp.dot(rrow, WA2T_ref[...], preferred_element_type=jnp.float32)
    Kr = jnp.dot(rrow, Wk2T_ref[...], preferred_element_type=jnp.float32)
    s_r = jnp.sum(rrow)
    ss_r = jnp.sum(rrow * rrow)

    s = stats_scr[:, 0:1]
    ss = stats_scr[:, 1:2]
    mu = (s + s_r) * (1.0 / 512.0)
    var = (ss + ss_r) * (1.0 / 512.0) - mu * mu
    inv = lax.rsqrt(var + EPS)

    a = (Ah_scr[...] + Ar - mu * sWA_ref[...]) * inv + bA_ref[...]
    x2 = jnp.where(a > 0.0, a, jnp.exp(jnp.minimum(a, 0.0)) - 1.0)
    hs2 = hb + jnp.dot(x2, WBT_ref[...],
                       preferred_element_type=jnp.float32) + bB_ref[...]
    K = jnp.dot(hs2, Wk1T_ref[...], preferred_element_type=jnp.float32) + Kr
    K_out[...] = K
    V_out[...] = hs2


def _dense_tables(hpad, relvectors, WA1T, WA2T, WBT, Wk1T, Wk2T, WqT,
                  sWA, bA_eff, bB):
    nblk = NPAD // BLK
    grid = (nblk, NUMRELS)
    full = lambda shape: pl.BlockSpec(shape, lambda b, r: (0, 0))
    out_shapes = (
        jax.ShapeDtypeStruct((NUMRELS * NPAD, HDIM), jnp.float32),
        jax.ShapeDtypeStruct((NUMRELS * NPAD, HDIM), jnp.float32),
        jax.ShapeDtypeStruct((NPAD, HDIM), jnp.float32),
    )
    return pl.pallas_call(
        _dense_body,
        grid=grid,
        in_specs=[
            pl.BlockSpec((BLK, HDIM), lambda b, r: (b, 0)),
            pl.BlockSpec((1, 1, HDIM), lambda b, r: (r, 0, 0)),
            full((HDIM, HDIM)),
            full((HDIM, HDIM)),
            full((HDIM, HDIM)),
            full((HDIM, HDIM)),
            full((HDIM, HDIM)),
            full((HDIM, HDIM)),
            full((1, HDIM)),
            full((1, HDIM)),
            full((1, HDIM)),
        ],
        out_specs=(
            pl.BlockSpec((BLK, HDIM), lambda b, r: (r * (NPAD // BLK) + b, 0)),
            pl.BlockSpec((BLK, HDIM), lambda b, r: (r * (NPAD // BLK) + b, 0)),
            pl.BlockSpec((BLK, HDIM), lambda b, r: (b, 0)),
        ),
        out_shape=out_shapes,
        scratch_shapes=[
            pltpu.VMEM((BLK, HDIM), jnp.float32),
            pltpu.VMEM((BLK, 8), jnp.float32),
        ],
    )(hpad, relvectors.reshape(NUMRELS, 1, HDIM), WA1T, WA2T, WBT, Wk1T,
      Wk2T, WqT, sWA, bA_eff, bB)


def _edge_body(ed_hbm, q_hbm, K_hbm, V_hbm, red_hbm,
               qbuf, acc, mden, edbufA, edbufB, pairbuf, lidxbuf,
               krows, vrows, cnt_ref, sem1, sem2, semA, semB):
    cc = lax.axis_index("c")
    ss = lax.axis_index("s")
    wid = ss * 2 + cc
    lane = lax.iota(jnp.int32, 16)

    def process_one(j, _):
        ld = lidxbuf[pl.ds(j, 16)][0]
        prods = []
        for t in range(16):
            qv = qbuf[ld, pl.ds(t * 16, 16)]
            kv = krows[j, pl.ds(t * 16, 16)]
            prods.append(qv * kv)
        wsc = []
        for hh in range(4):
            sv = (prods[4 * hh] + prods[4 * hh + 1]) + \
                 (prods[4 * hh + 2] + prods[4 * hh + 3])
            wsc.append(jnp.sum(sv) * 0.125)
        w_v = jnp.where(lane == 0, wsc[0],
              jnp.where(lane == 1, wsc[1],
              jnp.where(lane == 2, wsc[2], wsc[3])))
        m_v = mden[ld, 0, :]
        d_v = mden[ld, 1, :]
        m_n = jnp.maximum(m_v, w_v)
        scale = jnp.exp(m_v - m_n)
        p_v = jnp.exp(w_v - m_n)
        mden[ld, 0, :] = m_n
        mden[ld, 1, :] = d_v * scale + p_v
        for hh in range(4):
            s_s = scale[hh]
            p_s = p_v[hh]
            for t in range(4):
                col = hh * 64 + t * 16
                av = acc[ld, pl.ds(col, 16)]
                acc[ld, pl.ds(col, 16)] = (av * s_s
                                           + p_s * vrows[j, pl.ds(col, 16)])
        return 0

    def flush():
        n = cnt_ref[0]

        @pl.when(n > 0)
        def _():
            cp1 = pltpu.async_copy(K_hbm.at[pairbuf], krows, sem1)
            cp2 = pltpu.async_copy(V_hbm.at[pairbuf], vrows, sem2)
            cp1.wait()
            cp2.wait()
            lax.fori_loop(0, n, process_one, 0)
            cnt_ref[0] = 0

    def start_chunk(ci, edbuf, sem):
        pltpu.async_copy(ed_hbm.at[:, pl.ds(ci * EB, EB)], edbuf, sem)

    def wait_chunk(edbuf, sem):
        pltpu.make_async_copy(ed_hbm.at[:, pl.ds(0, EB)], edbuf, sem).wait()

    def scan_chunk(edbuf, n0):
        def grp_body(g, _):
            dv = edbuf[0, pl.ds(g * 16, 16)]
            lv = dv - n0
            mask = (lv >= 0) & (lv < NB)
            cnt = plsc.all_reduce_population_count(mask)[0]

            @pl.when(cnt > 0)
            def _():
                pv = edbuf[1, pl.ds(g * 16, 16)]
                cur = cnt_ref[0]
                plsc.store_compressed(pairbuf.at[pl.ds(cur, 16)], pv,
                                      mask=mask)
                plsc.store_compressed(lidxbuf.at[pl.ds(cur, 16)], lv,
                                      mask=mask)
                cnt_ref[0] = cur + cnt

                @pl.when(cur + cnt >= CAP)
                def _():
                    flush()

            return 0

        lax.fori_loop(0, NGRP, grp_body, 0, unroll=4)

    def do_round(rnd, _):
        n0 = (wid * ROUNDS + rnd) * NB
        zero16 = jnp.zeros((16,), jnp.float32)
        neg16 = jnp.full((16,), NEG, jnp.float32)

        def init_row(i, _):
            for t in range(16):
                acc[i, pl.ds(t * 16, 16)] = zero16
            mden[i, 0, :] = neg16
            mden[i, 1, :] = zero16
            return 0

        lax.fori_loop(0, NB, init_row, 0, unroll=4)
        for b in range(BUF // 16):
            pairbuf[pl.ds(b * 16, 16)] = jnp.zeros((16,), jnp.int32)
        cnt_ref[0] = 0
        pltpu.sync_copy(q_hbm.at[pl.ds(n0, NB)], qbuf)

        def pair_body(k2, _):
            c0 = k2 * 2
            wait_chunk(edbufA, semA)
            start_chunk(c0 + 1, edbufB, semB)
            scan_chunk(edbufA, n0)
            wait_chunk(edbufB, semB)
            nxt = jnp.where(c0 + 2 >= NCHUNK, 0, c0 + 2)
            start_chunk(nxt, edbufA, semA)
            scan_chunk(edbufB, n0)
            return 0

        lax.fori_loop(0, NCHUNK // 2, pair_body, 0, unroll=False)
        flush()

        def norm_row(i, _):
            d_v = mden[i, 1, :]
            inv_v = jnp.where(d_v > 0.0, 1.0 / d_v, 0.0)
            for hh in range(4):
                inv = inv_v[hh]
                for t in range(4):
                    col = hh * 64 + t * 16
                    acc[i, pl.ds(col, 16)] = acc[i, pl.ds(col, 16)] * inv
            return 0

        lax.fori_loop(0, NB, norm_row, 0, unroll=4)
        pltpu.sync_copy(acc, red_hbm.at[pl.ds(n0, NB)])
        return 0

    start_chunk(0, edbufA, semA)
    lax.fori_loop(0, ROUNDS, do_round, 0, unroll=False)
    wait_chunk(edbufA, semA)


def _edge_phase(ed, q, Kflat, Vflat):
    mesh = plsc.VectorSubcoreMesh(core_axis_name="c", subcore_axis_name="s")
    f = functools.partial(
        pl.kernel,
        out_type=jax.ShapeDtypeStruct((NPAD, HDIM), jnp.float32),
        mesh=mesh,
        compiler_params=pltpu.CompilerParams(needs_layout_passes=False,
                                             use_tc_tiling_on_sc=False),
        scratch_types=[
            pltpu.VMEM((NB, HDIM), jnp.float32),
            pltpu.VMEM((NB, HDIM), jnp.float32),
            pltpu.VMEM((NB, 2, 16), jnp.float32),
            pltpu.VMEM((2, EB), jnp.int32),
            pltpu.VMEM((2, EB), jnp.int32),
            pltpu.VMEM((BUF,), jnp.int32),
            pltpu.VMEM((BUF + 16,), jnp.int32),
            pltpu.VMEM((BUF, HDIM), jnp.float32),
            pltpu.VMEM((BUF, HDIM), jnp.float32),
            pltpu.SMEM((1,), jnp.int32),
            pltpu.SemaphoreType.DMA,
            pltpu.SemaphoreType.DMA,
            pltpu.SemaphoreType.DMA,
            pltpu.SemaphoreType.DMA,
        ],
    )(_edge_body)
    return f(ed, q, Kflat, Vflat)


def kernel(h, edge_index, edge_type, relvectors, ln_w, ln_b, WA, bA, WB, bB,
           Wq, Wk):
    src = edge_index[0]
    dst = edge_index[1]

    WA_p = WA * ln_w[None, :]
    bA_eff = (bA + WA @ ln_b)[None, :]
    sWA = jnp.sum(WA_p, axis=1)[None, :]
    WA1T = WA_p[:, :HDIM].T
    WA2T = WA_p[:, HDIM:].T
    Wk1T = Wk[:, :HDIM].T
    Wk2T = Wk[:, HDIM:].T
    WBT = WB.T
    WqT = Wq.T

    hpad = jnp.pad(h, ((0, NPAD - N), (0, 0)))

    Kflat, Vflat, q = _dense_tables(hpad, relvectors, WA1T, WA2T, WBT, Wk1T,
                                    Wk2T, WqT, sWA, bA_eff, bB[None, :])
    ed = jnp.stack([dst, edge_type * NPAD + src])

    red = _edge_phase(ed, q, Kflat, Vflat)
    return red[:N]


# speedup vs baseline: 2.3114x; 2.3114x over previous
import functools

import jax
import jax.numpy as jnp
from jax import lax
from jax.experimental import pallas as pl
from jax.experimental.pallas import tpu as pltpu
from jax.experimental.pallas import tpu_sc as plsc

HDIM = 256
NUMRELS = 16
NHEADS = 4
DH = HDIM // NHEADS
N = 10000
NPAD = 10240
E = 160000
BLK = 512
EPS = 1e-5

NB = 160
ROUNDS = 2
EB = 1600
NGRP = EB // 16
NCHUNK = E // EB
CAP = 48
BUF = 64
NEG = -1e30


def _dense_body(h_ref, relv_ref, WA1T_ref, WA2T_ref, WBT_ref, Wk1T_ref,
                Wk2T_ref, WqT_ref, sWA_ref, bA_ref, bB_ref,
                KV_out, q_out, Ah_scr, stats_scr):
    r = pl.program_id(1)
    hb = h_ref[...]

    @pl.when(r == 0)
    def _precompute():
        Ah_scr[...] = jnp.dot(hb, WA1T_ref[...],
                              preferred_element_type=jnp.float32)
        stats_scr[:, 0:1] = jnp.sum(hb, axis=1, keepdims=True)
        stats_scr[:, 1:2] = jnp.sum(hb * hb, axis=1, keepdims=True)
        q_out[...] = jnp.dot(hb, WqT_ref[...],
                             preferred_element_type=jnp.float32)

    rrow = relv_ref[0]
    Ar = jnp.dot(rrow, WA2T_ref[...], preferred_element_type=jnp.float32)
    Kr = jnp.dot(rrow, Wk2T_ref[...], preferred_element_type=jnp.float32)
    s_r = jnp.sum(rrow)
    ss_r = jnp.sum(rrow * rrow)

    s = stats_scr[:, 0:1]
    ss = stats_scr[:, 1:2]
    mu = (s + s_r) * (1.0 / 512.0)
    var = (ss + ss_r) * (1.0 / 512.0) - mu * mu
    inv = lax.rsqrt(var + EPS)

    a = (Ah_scr[...] + Ar - mu * sWA_ref[...]) * inv + bA_ref[...]
    x2 = jnp.where(a > 0.0, a, jnp.exp(jnp.minimum(a, 0.0)) - 1.0)
    hs2 = hb + jnp.dot(x2, WBT_ref[...],
                       preferred_element_type=jnp.float32) + bB_ref[...]
    K = jnp.dot(hs2, Wk1T_ref[...], preferred_element_type=jnp.float32) + Kr
    KV_out[:, :HDIM] = K
    KV_out[:, HDIM:] = hs2


def _dense_tables(hpad, relvectors, WA1T, WA2T, WBT, Wk1T, Wk2T, WqT,
                  sWA, bA_eff, bB):
    nblk = NPAD // BLK
    grid = (nblk, NUMRELS)
    full = lambda shape: pl.BlockSpec(shape, lambda b, r: (0, 0))
    out_shapes = (
        jax.ShapeDtypeStruct((NUMRELS * NPAD, 2 * HDIM), jnp.float32),
        jax.ShapeDtypeStruct((NPAD, HDIM), jnp.float32),
    )
    return pl.pallas_call(
        _dense_body,
        grid=grid,
        in_specs=[
            pl.BlockSpec((BLK, HDIM), lambda b, r: (b, 0)),
            pl.BlockSpec((1, 1, HDIM), lambda b, r: (r, 0, 0)),
            full((HDIM, HDIM)),
            full((HDIM, HDIM)),
            full((HDIM, HDIM)),
            full((HDIM, HDIM)),
            full((HDIM, HDIM)),
            full((HDIM, HDIM)),
            full((1, HDIM)),
            full((1, HDIM)),
            full((1, HDIM)),
        ],
        out_specs=(
            pl.BlockSpec((BLK, 2 * HDIM),
                         lambda b, r: (r * (NPAD // BLK) + b, 0)),
            pl.BlockSpec((BLK, HDIM), lambda b, r: (b, 0)),
        ),
        out_shape=out_shapes,
        scratch_shapes=[
            pltpu.VMEM((BLK, HDIM), jnp.float32),
            pltpu.VMEM((BLK, 8), jnp.float32),
        ],
    )(hpad, relvectors.reshape(NUMRELS, 1, HDIM), WA1T, WA2T, WBT, Wk1T,
      Wk2T, WqT, sWA, bA_eff, bB)


def _edge_body(ed_hbm, q_hbm, KV_hbm, red_hbm,
               qbuf, acc, mden, edbufA, edbufB, pairbuf, lidxbuf,
               kvrows, cnt_ref, sem1, semA, semB):
    cc = lax.axis_index("c")
    ss = lax.axis_index("s")
    wid = ss * 2 + cc
    lane = lax.iota(jnp.int32, 16)

    def process_one(j, _):
        ld = lidxbuf[pl.ds(j, 16)][0]
        prods = []
        for t in range(16):
            qv = qbuf[ld, pl.ds(t * 16, 16)]
            kv = kvrows[j, pl.ds(t * 16, 16)]
            prods.append(qv * kv)
        wsc = []
        for hh in range(4):
            sv = (prods[4 * hh] + prods[4 * hh + 1]) + \
                 (prods[4 * hh + 2] + prods[4 * hh + 3])
            wsc.append(jnp.sum(sv) * 0.125)
        w_v = jnp.where(lane == 0, wsc[0],
              jnp.where(lane == 1, wsc[1],
              jnp.where(lane == 2, wsc[2], wsc[3])))
        m_v = mden[ld, 0, :]
        d_v = mden[ld, 1, :]
        m_n = jnp.maximum(m_v, w_v)
        scale = jnp.exp(m_v - m_n)
        p_v = jnp.exp(w_v - m_n)
        mden[ld, 0, :] = m_n
        mden[ld, 1, :] = d_v * scale + p_v
        for hh in range(4):
            s_s = scale[hh]
            p_s = p_v[hh]
            for t in range(4):
                col = hh * 64 + t * 16
                av = acc[ld, pl.ds(col, 16)]
                acc[ld, pl.ds(col, 16)] = (
                    av * s_s + p_s * kvrows[j, pl.ds(HDIM + col, 16)])
        return 0

    def flush():
        n = cnt_ref[0]

        @pl.when(n > 0)
        def _():
            pltpu.async_copy(KV_hbm.at[pairbuf], kvrows, sem1).wait()
            lax.fori_loop(0, n, process_one, 0)
            cnt_ref[0] = 0

    def start_chunk(ci, edbuf, sem):
        pltpu.async_copy(ed_hbm.at[:, pl.ds(ci * EB, EB)], edbuf, sem)

    def wait_chunk(edbuf, sem):
        pltpu.make_async_copy(ed_hbm.at[:, pl.ds(0, EB)], edbuf, sem).wait()

    def scan_chunk(edbuf, n0):
        def grp_body(g, _):
            dv = edbuf[0, pl.ds(g * 16, 16)]
            lv = dv - n0
            mask = (lv >= 0) & (lv < NB)
            cnt = plsc.all_reduce_population_count(mask)[0]

            @pl.when(cnt > 0)
            def _():
                pv = edbuf[1, pl.ds(g * 16, 16)]
                cur = cnt_ref[0]
                plsc.store_compressed(pairbuf.at[pl.ds(cur, 16)], pv,
                                      mask=mask)
                plsc.store_compressed(lidxbuf.at[pl.ds(cur, 16)], lv,
                                      mask=mask)
                cnt_ref[0] = cur + cnt

                @pl.when(cur + cnt >= CAP)
                def _():
                    flush()

            return 0

        lax.fori_loop(0, NGRP, grp_body, 0, unroll=4)

    def do_round(rnd, _):
        n0 = (wid * ROUNDS + rnd) * NB
        zero16 = jnp.zeros((16,), jnp.float32)
        neg16 = jnp.full((16,), NEG, jnp.float32)

        def init_row(i, _):
            for t in range(16):
                acc[i, pl.ds(t * 16, 16)] = zero16
            mden[i, 0, :] = neg16
            mden[i, 1, :] = zero16
            return 0

        lax.fori_loop(0, NB, init_row, 0, unroll=4)
        for b in range(BUF // 16):
            pairbuf[pl.ds(b * 16, 16)] = jnp.zeros((16,), jnp.int32)
        cnt_ref[0] = 0
        pltpu.sync_copy(q_hbm.at[pl.ds(n0, NB)], qbuf)

        def pair_body(k2, _):
            c0 = k2 * 2
            wait_chunk(edbufA, semA)
            start_chunk(c0 + 1, edbufB, semB)
            scan_chunk(edbufA, n0)
            wait_chunk(edbufB, semB)
            nxt = jnp.where(c0 + 2 >= NCHUNK, 0, c0 + 2)
            start_chunk(nxt, edbufA, semA)
            scan_chunk(edbufB, n0)
            return 0

        lax.fori_loop(0, NCHUNK // 2, pair_body, 0, unroll=False)
        flush()

        def norm_row(i, _):
            d_v = mden[i, 1, :]
            inv_v = jnp.where(d_v > 0.0, 1.0 / d_v, 0.0)
            for hh in range(4):
                inv = inv_v[hh]
                for t in range(4):
                    col = hh * 64 + t * 16
                    acc[i, pl.ds(col, 16)] = acc[i, pl.ds(col, 16)] * inv
            return 0

        lax.fori_loop(0, NB, norm_row, 0, unroll=4)
        pltpu.sync_copy(acc, red_hbm.at[pl.ds(n0, NB)])
        return 0

    start_chunk(0, edbufA, semA)
    lax.fori_loop(0, ROUNDS, do_round, 0, unroll=False)
    wait_chunk(edbufA, semA)


def _edge_phase(ed, q, KVflat):
    mesh = plsc.VectorSubcoreMesh(core_axis_name="c", subcore_axis_name="s")
    f = functools.partial(
        pl.kernel,
        out_type=jax.ShapeDtypeStruct((NPAD, HDIM), jnp.float32),
        mesh=mesh,
        compiler_params=pltpu.CompilerParams(needs_layout_passes=False,
                                             use_tc_tiling_on_sc=False),
        scratch_types=[
            pltpu.VMEM((NB, HDIM), jnp.float32),
            pltpu.VMEM((NB, HDIM), jnp.float32),
            pltpu.VMEM((NB, 2, 16), jnp.float32),
            pltpu.VMEM((2, EB), jnp.int32),
            pltpu.VMEM((2, EB), jnp.int32),
            pltpu.VMEM((BUF,), jnp.int32),
            pltpu.VMEM((BUF + 16,), jnp.int32),
            pltpu.VMEM((BUF, 2 * HDIM), jnp.float32),
            pltpu.SMEM((1,), jnp.int32),
            pltpu.SemaphoreType.DMA,
            pltpu.SemaphoreType.DMA,
            pltpu.SemaphoreType.DMA,
        ],
    )(_edge_body)
    return f(ed, q, KVflat)


def kernel(h, edge_index, edge_type, relvectors, ln_w, ln_b, WA, bA, WB, bB,
           Wq, Wk):
    src = edge_index[0]
    dst = edge_index[1]

    WA_p = WA * ln_w[None, :]
    bA_eff = (bA + WA @ ln_b)[None, :]
    sWA = jnp.sum(WA_p, axis=1)[None, :]
    WA1T = WA_p[:, :HDIM].T
    WA2T = WA_p[:, HDIM:].T
    Wk1T = Wk[:, :HDIM].T
    Wk2T = Wk[:, HDIM:].T
    WBT = WB.T
    WqT = Wq.T

    hpad = jnp.pad(h, ((0, NPAD - N), (0, 0)))

    KVflat, q = _dense_tables(hpad, relvectors, WA1T, WA2T, WBT, Wk1T,
                              Wk2T, WqT, sWA, bA_eff, bB[None, :])
    ed = jnp.stack([dst, edge_type * NPAD + src])

    red = _edge_phase(ed, q, KVflat)
    return red[:N]
